# Initial kernel scaffold; baseline (speedup 1.0000x reference)
import jax
import jax.numpy as jnp
from jax.experimental import pallas as pl


def kernel(feat, edge_index, efeat, W_edge, b_edge, W_ih, W_hh, b_ih, b_hh):
    raise NotImplementedError("write your pallas kernel here")


# speedup vs baseline: 1.2819x; 1.2819x over previous
import functools

import jax
import jax.numpy as jnp
from jax import lax
from jax.experimental import pallas as pl
from jax.experimental.pallas import tpu as pltpu
from jax.experimental.pallas import tpu_sc as plsc

N = 10000
E = 320000
F = 16
NC, NS = 2, 16
NW = NC * NS
EPW = E // NW
CH = 128
NFULL = EPW // CH
TAIL = EPW - NFULL * CH
GK = 6
NG = NFULL // GK
ROWS_PW = N // NS

_f32 = jnp.float32


def _mesh():
    return plsc.VectorSubcoreMesh(core_axis_name="c", subcore_axis_name="s")


_SC_PARAMS = pltpu.CompilerParams(use_tc_tiling_on_sc=False)


def _gather_body(h_hbm, src_hbm, out_hbm, idx_all, rows0, rows1, trows,
                 sem0, sem1, semt):
    wid = lax.axis_index("s") * NC + lax.axis_index("c")
    base = wid * EPW
    pltpu.sync_copy(src_hbm.at[pl.ds(base, EPW)], idx_all)

    rows = (rows0, rows1)
    sems = (sem0, sem1)
    pending = [None, None]

    def fire(g):
        p = g % 2
        descs = []
        for j in range(GK):
            k = g * GK + j
            descs.append(pltpu.async_copy(
                h_hbm.at[idx_all.at[pl.ds(k * CH, CH)]],
                rows[p].at[pl.ds(j * CH, CH)],
                sems[p]))
        pending[p] = descs

    def drain(g):
        p = g % 2
        for d in pending[p]:
            d.wait()
        pltpu.sync_copy(rows[p],
                        out_hbm.at[pl.ds(base + g * GK * CH, GK * CH)])

    fire(0)
    for g in range(1, NG):
        fire(g)
        drain(g - 1)
    drain(NG - 1)
    pltpu.async_copy(h_hbm.at[idx_all.at[pl.ds(NFULL * CH, TAIL)]],
                     trows, semt).wait()
    pltpu.sync_copy(trows, out_hbm.at[pl.ds(base + NFULL * CH, TAIL)])


def _sc_gather(h, src):
    fn = pl.kernel(
        _gather_body,
        out_type=jax.ShapeDtypeStruct((E, F), _f32),
        mesh=_mesh(),
        scratch_types=[
            pltpu.VMEM((EPW,), jnp.int32),
            pltpu.VMEM((GK * CH, F), _f32),
            pltpu.VMEM((GK * CH, F), _f32),
            pltpu.VMEM((TAIL, F), _f32),
            pltpu.SemaphoreType.DMA,
            pltpu.SemaphoreType.DMA,
            pltpu.SemaphoreType.DMA,
        ],
        compiler_params=_SC_PARAMS,
    )
    return fn(h, src)


def _scatter_body(m_hbm, dst_hbm, zeros_hbm, out_hbm,
                  idx0, idx1, m0, m1, idxt, mt, shared, sem0, sem1, semt):
    c = lax.axis_index("c")
    s = lax.axis_index("s")
    wid = s * NC + c
    base = wid * EPW

    pltpu.sync_copy(zeros_hbm.at[pl.ds(s * ROWS_PW, ROWS_PW)],
                    shared.at[pl.ds(s * ROWS_PW, ROWS_PW)])
    plsc.subcore_barrier()

    idxb = (idx0, idx1)
    mb = (m0, m1)
    sems = (sem0, sem1)
    pending = [None, None]

    def fire(j):
        p = j % 2
        d1 = pltpu.async_copy(dst_hbm.at[pl.ds(base + j * CH, CH)],
                              idxb[p], sems[p])
        d2 = pltpu.async_copy(m_hbm.at[pl.ds(base + j * CH, CH)],
                              mb[p], sems[p])
        pending[p] = (d1, d2)

    fire(0)
    for j in range(NFULL):
        p = j % 2
        if j + 1 < NFULL:
            fire(j + 1)
        for d in pending[p]:
            d.wait()
        pltpu.sync_copy(mb[p], shared.at[idxb[p]], add=True)
    d1 = pltpu.async_copy(dst_hbm.at[pl.ds(base + NFULL * CH, TAIL)],
                          idxt, semt)
    d2 = pltpu.async_copy(m_hbm.at[pl.ds(base + NFULL * CH, TAIL)],
                          mt, semt)
    d1.wait()
    d2.wait()
    pltpu.sync_copy(mt, shared.at[idxt], add=True)

    plsc.subcore_barrier()
    pltpu.sync_copy(shared.at[pl.ds(s * ROWS_PW, ROWS_PW)],
                    out_hbm.at[pl.ds(c * N + s * ROWS_PW, ROWS_PW)])


def _sc_scatter(m, dst, zeros):
    fn = pl.kernel(
        _scatter_body,
        out_type=jax.ShapeDtypeStruct((NC * N, F), _f32),
        mesh=_mesh(),
        scratch_types=[
            pltpu.VMEM((CH,), jnp.int32),
            pltpu.VMEM((CH,), jnp.int32),
            pltpu.VMEM((CH, F), _f32),
            pltpu.VMEM((CH, F), _f32),
            pltpu.VMEM((TAIL,), jnp.int32),
            pltpu.VMEM((TAIL, F), _f32),
            pltpu.VMEM_SHARED((N, F), _f32),
            pltpu.SemaphoreType.DMA,
            pltpu.SemaphoreType.DMA,
            pltpu.SemaphoreType.DMA,
        ],
        compiler_params=_SC_PARAMS,
    )
    return fn(m, dst, zeros)


_BM = 4000


def _msg_body(ef_ref, hs_ref, we_ref, be_ref, m_ref):
    w = jnp.dot(ef_ref[...], we_ref[...],
                preferred_element_type=_f32) + be_ref[...]
    hs = hs_ref[...]
    acc = hs[:, 0:1] * w[:, 0:F]
    for i in range(1, F):
        acc = acc + hs[:, i:i + 1] * w[:, i * F:(i + 1) * F]
    m_ref[...] = acc


def _tc_messages(efeat, h_src, W_edge, b_edge2):
    grid = (E // _BM,)
    return pl.pallas_call(
        _msg_body,
        grid=grid,
        in_specs=[
            pl.BlockSpec((_BM, F), lambda i: (i, 0)),
            pl.BlockSpec((_BM, F), lambda i: (i, 0)),
            pl.BlockSpec((F, F * F), lambda i: (0, 0)),
            pl.BlockSpec((1, F * F), lambda i: (0, 0)),
        ],
        out_specs=pl.BlockSpec((_BM, F), lambda i: (i, 0)),
        out_shape=jax.ShapeDtypeStruct((E, F), _f32),
    )(efeat, h_src, W_edge, b_edge2)


def _gru_body(rst_ref, h_ref, wi_ref, wh_ref, bi_ref, bh_ref, out_ref):
    x = rst_ref[0:N, :] + rst_ref[N:2 * N, :]
    h = h_ref[...]
    gi = jnp.dot(x, wi_ref[...], preferred_element_type=_f32) + bi_ref[...]
    gh = jnp.dot(h, wh_ref[...], preferred_element_type=_f32) + bh_ref[...]
    r = jax.nn.sigmoid(gi[:, 0:F] + gh[:, 0:F])
    z = jax.nn.sigmoid(gi[:, F:2 * F] + gh[:, F:2 * F])
    n = jnp.tanh(gi[:, 2 * F:3 * F] + r * gh[:, 2 * F:3 * F])
    out_ref[...] = (1.0 - z) * n + z * h


def _tc_gru(rst2, h, W_ihT, W_hhT, b_ih2, b_hh2):
    return pl.pallas_call(
        _gru_body,
        out_shape=jax.ShapeDtypeStruct((N, F), _f32),
    )(rst2, h, W_ihT, W_hhT, b_ih2, b_hh2)


@jax.jit
def kernel(feat, edge_index, efeat, W_edge, b_edge, W_ih, W_hh, b_ih, b_hh):
    src = edge_index[0]
    dst = edge_index[1]
    b_edge2 = b_edge[None, :]
    W_ihT = W_ih.T
    W_hhT = W_hh.T
    b_ih2 = b_ih[None, :]
    b_hh2 = b_hh[None, :]
    zeros = jnp.zeros((N, F), _f32)

    h = feat
    for _ in range(2):
        h_src = _sc_gather(h, src)
        m = _tc_messages(efeat, h_src, W_edge, b_edge2)
        rst2 = _sc_scatter(m, dst, zeros)
        h = _tc_gru(rst2, h, W_ihT, W_hhT, b_ih2, b_hh2)
    return h


# speedup vs baseline: 6.9290x; 5.4050x over previous
import jax
import jax.numpy as jnp
from jax import lax
from jax.experimental import pallas as pl
from jax.experimental.pallas import tpu as pltpu
from jax.experimental.pallas import tpu_sc as plsc

N = 10000
E = 320000
F = 16
NC, NS = 2, 16
NW = NC * NS
EPW = E // NW
CH = 128
NFULL = EPW // CH
TAIL = EPW - NFULL * CH
GK = 6
NG = NFULL // GK
EP = E // 8
NP = N // 8

_f32 = jnp.float32


def _mesh():
    return plsc.VectorSubcoreMesh(core_axis_name="c", subcore_axis_name="s")


_SC_PARAMS = pltpu.CompilerParams(use_tc_tiling_on_sc=False)


def _gather_body(h_hbm, src_hbm, out_hbm, idx_all, rows0, rows1, trows,
                 sem0, sem1, semt):
    wid = lax.axis_index("s") * NC + lax.axis_index("c")
    base = wid * EPW
    pltpu.sync_copy(src_hbm.at[pl.ds(base, EPW)], idx_all)

    rows = (rows0, rows1)
    sems = (sem0, sem1)
    pending = [None, None]

    def fire(g):
        p = g % 2
        descs = []
        for j in range(GK):
            k = g * GK + j
            descs.append(pltpu.async_copy(
                h_hbm.at[idx_all.at[pl.ds(k * CH, CH)]],
                rows[p].at[pl.ds(j * CH, CH)],
                sems[p]))
        pending[p] = descs

    def drain(g):
        p = g % 2
        for d in pending[p]:
            d.wait()
        pltpu.sync_copy(rows[p],
                        out_hbm.at[pl.ds(base + g * GK * CH, GK * CH)])

    fire(0)
    for g in range(1, NG):
        fire(g)
        drain(g - 1)
    drain(NG - 1)
    pltpu.async_copy(h_hbm.at[idx_all.at[pl.ds(NFULL * CH, TAIL)]],
                     trows, semt).wait()
    pltpu.sync_copy(trows, out_hbm.at[pl.ds(base + NFULL * CH, TAIL)])


def _sc_gather(h_tab, src):
    fn = pl.kernel(
        _gather_body,
        out_type=jax.ShapeDtypeStruct((E, F), _f32),
        mesh=_mesh(),
        scratch_types=[
            pltpu.VMEM((EPW,), jnp.int32),
            pltpu.VMEM((GK * CH, F), _f32),
            pltpu.VMEM((GK * CH, F), _f32),
            pltpu.VMEM((TAIL, F), _f32),
            pltpu.SemaphoreType.DMA,
            pltpu.SemaphoreType.DMA,
            pltpu.SemaphoreType.DMA,
        ],
        compiler_params=_SC_PARAMS,
    )
    return fn(h_tab, src)


_ROWS_PW = N // NS


def _scatter_body(m_hbm, dst_hbm, zeros_hbm, out_hbm,
                  idx0, idx1, m0, m1, idxt, mt, shared, sem0, sem1, semt):
    c = lax.axis_index("c")
    s = lax.axis_index("s")
    wid = s * NC + c
    base = wid * EPW

    pltpu.sync_copy(zeros_hbm.at[pl.ds(s * _ROWS_PW, _ROWS_PW)],
                    shared.at[pl.ds(s * _ROWS_PW, _ROWS_PW)])
    plsc.subcore_barrier()

    idxb = (idx0, idx1)
    mb = (m0, m1)
    sems = (sem0, sem1)
    pending = [None, None]

    def fire(j):
        p = j % 2
        d1 = pltpu.async_copy(dst_hbm.at[pl.ds(base + j * CH, CH)],
                              idxb[p], sems[p])
        d2 = pltpu.async_copy(m_hbm.at[pl.ds(base + j * CH, CH)],
                              mb[p], sems[p])
        pending[p] = (d1, d2)

    fire(0)
    for j in range(NFULL):
        p = j % 2
        if j + 1 < NFULL:
            fire(j + 1)
        for d in pending[p]:
            d.wait()
        pltpu.sync_copy(mb[p], shared.at[idxb[p]], add=True)
    d1 = pltpu.async_copy(dst_hbm.at[pl.ds(base + NFULL * CH, TAIL)],
                          idxt, semt)
    d2 = pltpu.async_copy(m_hbm.at[pl.ds(base + NFULL * CH, TAIL)],
                          mt, semt)
    d1.wait()
    d2.wait()
    pltpu.sync_copy(mt, shared.at[idxt], add=True)

    plsc.subcore_barrier()
    pltpu.sync_copy(shared.at[pl.ds(s * _ROWS_PW, _ROWS_PW)],
                    out_hbm.at[pl.ds(c * N + s * _ROWS_PW, _ROWS_PW)])


def _sc_scatter(m, dst, zeros):
    fn = pl.kernel(
        _scatter_body,
        out_type=jax.ShapeDtypeStruct((NC * N, F), _f32),
        mesh=_mesh(),
        scratch_types=[
            pltpu.VMEM((CH,), jnp.int32),
            pltpu.VMEM((CH,), jnp.int32),
            pltpu.VMEM((CH, F), _f32),
            pltpu.VMEM((CH, F), _f32),
            pltpu.VMEM((TAIL,), jnp.int32),
            pltpu.VMEM((TAIL, F), _f32),
            pltpu.VMEM_SHARED((N, F), _f32),
            pltpu.SemaphoreType.DMA,
            pltpu.SemaphoreType.DMA,
            pltpu.SemaphoreType.DMA,
        ],
        compiler_params=_SC_PARAMS,
    )
    return fn(m, dst, zeros)


_BM = 8000
_BMP = _BM // 8


def _msg_body(ef_ref, hs_ref, wz_ref, bb_ref, r_ref, t_ref, m_ref):
    ef = ef_ref[...]
    hs = hs_ref[...]
    ef_rep = jnp.dot(ef, r_ref[...], preferred_element_type=_f32)
    hs_tile = jnp.dot(hs, t_ref[...], preferred_element_type=_f32)
    z = ef_rep * hs_tile
    m_ref[...] = (jnp.dot(z, wz_ref[...], preferred_element_type=_f32)
                  + jnp.dot(hs, bb_ref[...], preferred_element_type=_f32))


def _tc_messages(efeat128, h_src128, WzB, BbB, RexpB, TexpB):
    grid = (E // _BM,)
    return pl.pallas_call(
        _msg_body,
        grid=grid,
        in_specs=[
            pl.BlockSpec((_BMP, 128), lambda i: (i, 0)),
            pl.BlockSpec((_BMP, 128), lambda i: (i, 0)),
            pl.BlockSpec((8 * F * F, 128), lambda i: (0, 0)),
            pl.BlockSpec((128, 128), lambda i: (0, 0)),
            pl.BlockSpec((128, 8 * F * F), lambda i: (0, 0)),
            pl.BlockSpec((128, 8 * F * F), lambda i: (0, 0)),
        ],
        out_specs=pl.BlockSpec((_BMP, 128), lambda i: (i, 0)),
        out_shape=jax.ShapeDtypeStruct((EP, 128), _f32),
    )(efeat128, h_src128, WzB, BbB, RexpB, TexpB)


def _gru_body(rst_ref, h_ref, wi_ref, wh_ref, bi_ref, bh_ref, out_ref):
    x = rst_ref[0:NP, :] + rst_ref[NP:2 * NP, :]
    h = h_ref[...]
    gi = jnp.dot(x, wi_ref[...], preferred_element_type=_f32) + bi_ref[...]
    gh = jnp.dot(h, wh_ref[...], preferred_element_type=_f32) + bh_ref[...]
    r = jax.nn.sigmoid(gi[:, 0:128] + gh[:, 0:128])
    z = jax.nn.sigmoid(gi[:, 128:256] + gh[:, 128:256])
    n = jnp.tanh(gi[:, 256:384] + r * gh[:, 256:384])
    out_ref[...] = (1.0 - z) * n + z * h


def _tc_gru(rst2, h128, W3i, W3h, b3i, b3h):
    return pl.pallas_call(
        _gru_body,
        out_shape=jax.ShapeDtypeStruct((NP, 128), _f32),
    )(rst2, h128, W3i, W3h, b3i, b3h)


def _pack_gru_w(WT):
    eye8 = jnp.eye(8, dtype=_f32)
    blocks = [jnp.kron(eye8, WT[:, g * F:(g + 1) * F]) for g in range(3)]
    return jnp.concatenate(blocks, axis=1)


def _pack_gru_b(b):
    return jnp.concatenate(
        [jnp.tile(b[g * F:(g + 1) * F], 8) for g in range(3)])[None, :]


@jax.jit
def kernel(feat, edge_index, efeat, W_edge, b_edge, W_ih, W_hh, b_ih, b_hh):
    src = edge_index[0]
    dst = edge_index[1]
    eye8 = jnp.eye(8, dtype=_f32)
    Wz = W_edge.reshape(F * F, F)
    Bb = b_edge.reshape(F, F)
    Rexp = jnp.repeat(jnp.eye(F, dtype=_f32), F, axis=1)
    Texp = jnp.tile(jnp.eye(F, dtype=_f32), (1, F))
    WzB = jnp.kron(eye8, Wz)
    BbB = jnp.kron(eye8, Bb)
    RexpB = jnp.kron(eye8, Rexp)
    TexpB = jnp.kron(eye8, Texp)
    W3i = _pack_gru_w(W_ih.T)
    W3h = _pack_gru_w(W_hh.T)
    b3i = _pack_gru_b(b_ih)
    b3h = _pack_gru_b(b_hh)
    zeros = jnp.zeros((N, F), _f32)
    efeat128 = efeat.reshape(EP, 128)

    h128 = feat.reshape(NP, 128)
    for _ in range(2):
        h_tab = h128.reshape(N, F)
        h_src = _sc_gather(h_tab, src)
        m128 = _tc_messages(efeat128, h_src.reshape(EP, 128),
                            WzB, BbB, RexpB, TexpB)
        rst2 = _sc_scatter(m128.reshape(E, F), dst, zeros)
        h128 = _tc_gru(rst2.reshape(2 * NP, 128), h128, W3i, W3h, b3i, b3h)
    return h128.reshape(N, F)
